# baseline (speedup 1.0000x reference)
import functools

import jax
import jax.numpy as jnp
from jax.experimental import pallas as pl

D_IN = 768
D_OUT = 768
LORA_R = 64
N_EXP = 64
TOKENS = 2048
TILE_T = 256


def _fused_body(x_ref, wb_ref, bb_ref, a_ref, rw_ref, b2_ref, o_ref):
    xt = x_ref[...]
    xb = xt.astype(jnp.bfloat16)

    logits = jax.lax.dot_general(
        xb, rw_ref[...], (((1,), (1,)), ((), ())),
        preferred_element_type=jnp.float32)
    m = jnp.max(logits, axis=1, keepdims=True)
    p = jnp.exp(logits - m)
    p = p / jnp.sum(p, axis=1, keepdims=True)

    lane = jax.lax.broadcasted_iota(jnp.int32, p.shape, 1)
    m1 = jnp.max(p, axis=1, keepdims=True)
    i1 = jnp.min(jnp.where(p == m1, lane, N_EXP), axis=1, keepdims=True)
    p_ex = jnp.where(lane == i1, -1.0, p)
    m2 = jnp.max(p_ex, axis=1, keepdims=True)
    i2 = jnp.min(jnp.where(p_ex == m2, lane, N_EXP), axis=1, keepdims=True)
    keep = (lane == i1) | (lane == i2)
    wv = jnp.where(keep, p, 0.0) / (m1 + m2 + 1e-6)

    h = jax.lax.dot_general(
        xb, a_ref[...], (((1,), (1,)), ((), ())),
        preferred_element_type=jnp.float32)

    pmat = (wv[:, :, None] * h[:, None, :]).reshape(
        TILE_T, N_EXP * LORA_R).astype(jnp.bfloat16)
    delta = jax.lax.dot_general(
        pmat, b2_ref[...], (((1,), (0,)), ((), ())),
        preferred_element_type=jnp.float32)

    base = jax.lax.dot_general(
        xb, wb_ref[...], (((1,), (1,)), ((), ())),
        preferred_element_type=jnp.float32)
    o_ref[...] = base + bb_ref[...] + delta


@functools.partial(jax.jit, static_argnames=("interpret",))
def kernel(x, W_base, b_base, A, B, router_w, interpret=False):
    t = x.shape[1]
    x2 = x.reshape(t, D_IN)
    b2 = B.astype(jnp.bfloat16).transpose(0, 2, 1).reshape(N_EXP * LORA_R, D_OUT)
    out = pl.pallas_call(
        _fused_body,
        grid=(t // TILE_T,),
        in_specs=[
            pl.BlockSpec((TILE_T, D_IN), lambda i: (i, 0)),
            pl.BlockSpec((D_OUT, D_IN), lambda i: (0, 0)),
            pl.BlockSpec((1, D_OUT), lambda i: (0, 0)),
            pl.BlockSpec((LORA_R, D_IN), lambda i: (0, 0)),
            pl.BlockSpec((N_EXP, D_IN), lambda i: (0, 0)),
            pl.BlockSpec((N_EXP * LORA_R, D_OUT), lambda i: (0, 0)),
        ],
        out_specs=pl.BlockSpec((TILE_T, D_OUT), lambda i: (i, 0)),
        out_shape=jax.ShapeDtypeStruct((t, D_OUT), jnp.float32),
        interpret=interpret,
    )(x2, W_base.astype(jnp.bfloat16), b_base.reshape(1, D_OUT),
      A.astype(jnp.bfloat16), router_w.astype(jnp.bfloat16), b2)
    return out.reshape(1, t, D_OUT)
